# Initial kernel scaffold; baseline (speedup 1.0000x reference)
import jax
import jax.numpy as jnp
from jax.experimental import pallas as pl


def kernel(x, W1, b1, W2, b2, Wl, bl, edge_index, batch):
    raise NotImplementedError("write your pallas kernel here")


# speedup vs baseline: 8.4657x; 8.4657x over previous
import functools

import jax
import jax.numpy as jnp
from jax import lax
from jax.experimental import pallas as pl
from jax.experimental.pallas import tpu as pltpu
from jax.experimental.pallas import tpu_sc as plsc

N = 10000
E = 160000
D = 256
H = 128
G = 64
NC = 2
NS = 16
R_MAIN = 624
R_EXTRA_OFF = NS * R_MAIN
R_EXTRA = N - R_EXTRA_OFF
EDGES_PER_TILE = E // NS
CH = 80
N_CHUNKS = EDGES_PER_TILE // CH
DEG_EDGES_PER_SC = E // NC
DEG_EDGES_PER_TILE = DEG_EDGES_PER_SC // NS
DCH = 40
N_DEG_CHUNKS = DEG_EDGES_PER_TILE // DCH
RB = 1000
N_RB = N // RB

_sc_mesh = plsc.VectorSubcoreMesh(core_axis_name="c", subcore_axis_name="s")


NP = 10240
DEG_E_TILE = E // (NC * NS)
COLS = 640


def _deg_body(dst_hbm, deg_out, dstbuf_v, hist_v, tmp_v, red_v, hist_sh):
    c = lax.axis_index("c")
    s = lax.axis_index("s")
    wid = c * NS + s

    def initz(i, _):
        hist_v[pl.ds(i * 16, 16)] = jnp.zeros((16,), jnp.float32)
        return _

    lax.fori_loop(0, NP // 16, initz, None)

    dstbuf_v[pl.ds((DEG_E_TILE // 16) * 16, 16)] = jnp.full((16,), N, jnp.int32)
    pltpu.sync_copy(dst_hbm.at[pl.ds(wid * DEG_E_TILE, DEG_E_TILE)],
                    dstbuf_v.at[pl.ds(0, DEG_E_TILE)])
    ones16 = jnp.ones((16,), jnp.float32)

    def scat(j, _):
        idx = dstbuf_v[pl.ds(j * 16, 16)]
        plsc.addupdate_scatter(hist_v, [idx], ones16)
        return _

    lax.fori_loop(0, DEG_E_TILE // 16 + 1, scat, None)

    pltpu.sync_copy(hist_v, hist_sh.at[s, 0])
    plsc.subcore_barrier()

    def initr(i, _):
        red_v[pl.ds(i * 16, 16)] = jnp.zeros((16,), jnp.float32)
        return _

    lax.fori_loop(0, COLS // 16, initr, None)

    def redk(k, _):
        pltpu.sync_copy(hist_sh.at[k, 0, pl.ds(s * COLS, COLS)], tmp_v)

        def add16(i, __):
            red_v[pl.ds(i * 16, 16)] += tmp_v[pl.ds(i * 16, 16)]
            return __

        lax.fori_loop(0, COLS // 16, add16, None)
        return _

    lax.fori_loop(0, NS, redk, None)

    pltpu.sync_copy(red_v, deg_out.at[pl.ds(c * NP + s * COLS, COLS)])


_deg_call = pl.kernel(
    _deg_body,
    out_type=jax.ShapeDtypeStruct((NC * NP,), jnp.float32),
    mesh=_sc_mesh,
    scratch_types=[
        pltpu.VMEM((DEG_E_TILE + 8,), jnp.int32),
        pltpu.VMEM((NP,), jnp.float32),
        pltpu.VMEM((COLS,), jnp.float32),
        pltpu.VMEM((COLS,), jnp.float32),
        pltpu.VMEM_SHARED((NS, 1, NP), jnp.float32),
    ],
    compiler_params=pltpu.CompilerParams(needs_layout_passes=False),
)


def _spmm_body(tbl_hbm, src2_hbm, dst_hbm, out_hbm, src_v, dst_v, rows_v, sem,
               acc_sh):
    c = lax.axis_index("c")
    s = lax.axis_index("s")

    if True:
        row0 = s * R_MAIN
        pltpu.sync_copy(tbl_hbm.at[pl.ds(c * N + row0, R_MAIN)],
                        acc_sh.at[pl.ds(row0, R_MAIN)])

        @pl.when(s == NS - 1)
        def _():
            pltpu.sync_copy(tbl_hbm.at[pl.ds(c * N + R_EXTRA_OFF, R_EXTRA)],
                            acc_sh.at[pl.ds(R_EXTRA_OFF, R_EXTRA)])

        plsc.subcore_barrier()

        ebase = s * EDGES_PER_TILE

        def chunk(j, _):
            base = ebase + j * CH
            pltpu.sync_copy(src2_hbm.at[pl.ds(c * E + base, CH)], src_v)
            pltpu.sync_copy(dst_hbm.at[pl.ds(base, CH)], dst_v)
            pltpu.async_copy(tbl_hbm.at[src_v], rows_v, sem).wait()
            pltpu.sync_copy(rows_v, acc_sh.at[dst_v], add=True)
            return _

        lax.fori_loop(0, N_CHUNKS, chunk, None)
        plsc.subcore_barrier()
        pltpu.sync_copy(acc_sh.at[pl.ds(row0, R_MAIN)],
                        out_hbm.at[c, pl.ds(row0, R_MAIN)])

        @pl.when(s == NS - 1)
        def _():
            pltpu.sync_copy(acc_sh.at[pl.ds(R_EXTRA_OFF, R_EXTRA)],
                            out_hbm.at[c, pl.ds(R_EXTRA_OFF, R_EXTRA)])


_spmm_call = pl.kernel(
    _spmm_body,
    out_type=jax.ShapeDtypeStruct((NC, N, H), jnp.float32),
    mesh=_sc_mesh,
    scratch_types=[
        pltpu.VMEM((CH,), jnp.int32),
        pltpu.VMEM((CH,), jnp.int32),
        pltpu.VMEM((CH, H), jnp.float32),
        pltpu.SemaphoreType.DMA,
        pltpu.VMEM_SHARED((N, H), jnp.float32),
    ],
)


def _dinv_block(deg_ref):
    degsum = deg_ref[:, 0:1] + deg_ref[:, 1:2] + 1.0
    return lax.rsqrt(degsum)


def _mm1_body(x_ref, w_ref, deg_ref, o_ref):
    dinv = _dinv_block(deg_ref)
    t = jnp.dot(x_ref[...], w_ref[...], preferred_element_type=jnp.float32)
    o_ref[0] = t * dinv


_mm1_call = pl.pallas_call(
    _mm1_body,
    grid=(N_RB, 2),
    in_specs=[
        pl.BlockSpec((RB, D), lambda i, j: (i, 0)),
        pl.BlockSpec((D, H), lambda i, j: (0, j)),
        pl.BlockSpec((RB, NC), lambda i, j: (i, 0)),
    ],
    out_specs=pl.BlockSpec((1, RB, H), lambda i, j: (j, i, 0)),
    out_shape=jax.ShapeDtypeStruct((NC, N, H), jnp.float32),
)


def _mm2_body(acc_ref, w_ref, b_ref, deg_ref, o_ref):
    dinv = _dinv_block(deg_ref)
    h1a = jax.nn.relu(acc_ref[0] * dinv + b_ref[0, 0][None, :])
    h1b = jax.nn.relu(acc_ref[1] * dinv + b_ref[0, 1][None, :])
    t = (jnp.dot(h1a, w_ref[0, 0], preferred_element_type=jnp.float32)
         + jnp.dot(h1b, w_ref[0, 1], preferred_element_type=jnp.float32))
    o_ref[0] = t * dinv


_mm2_call = pl.pallas_call(
    _mm2_body,
    grid=(N_RB, 2),
    in_specs=[
        pl.BlockSpec((NC, RB, H), lambda i, j: (0, i, 0)),
        pl.BlockSpec((1, NC, H, H), lambda i, j: (0, 0, 0, j)),
        pl.BlockSpec((1, NC, H), lambda i, j: (0, 0, 0)),
        pl.BlockSpec((RB, NC), lambda i, j: (i, 0)),
    ],
    out_specs=pl.BlockSpec((1, RB, H), lambda i, j: (j, i, 0)),
    out_shape=jax.ShapeDtypeStruct((NC, N, H), jnp.float32),
)


def _pool_body(acc_ref, b_ref, deg_ref, batch_ref, wl_ref, bl_ref, o_ref,
               pooled_acc, cnt_acc):
    i = pl.program_id(0)

    @pl.when(i == 0)
    def _():
        pooled_acc[...] = jnp.zeros_like(pooled_acc)
        cnt_acc[...] = jnp.zeros_like(cnt_acc)

    dinv = _dinv_block(deg_ref)
    h2a = jax.nn.relu(acc_ref[0] * dinv + b_ref[0, 0][None, :])
    h2b = jax.nn.relu(acc_ref[1] * dinv + b_ref[0, 1][None, :])
    h2 = jnp.concatenate([h2a, h2b], axis=1)
    bvec = batch_ref[0, 0, :]
    iota = lax.broadcasted_iota(jnp.int32, (RB, G), 1)
    onehot = (bvec[:, None] == iota).astype(jnp.float32)
    pooled_acc[...] += lax.dot_general(
        onehot, h2, (((0,), (0,)), ((), ())),
        preferred_element_type=jnp.float32)
    cnt_acc[...] += jnp.sum(onehot, axis=0, keepdims=True)

    @pl.when(i == N_RB - 1)
    def _():
        cnt = jnp.maximum(cnt_acc[0, :], 1.0)
        pooled = pooled_acc[...] / cnt[:, None]
        o_ref[...] = (jnp.dot(pooled, wl_ref[...],
                              preferred_element_type=jnp.float32)
                      + bl_ref[0, :][None, :])


_pool_call = pl.pallas_call(
    _pool_body,
    grid=(N_RB,),
    in_specs=[
        pl.BlockSpec((NC, RB, H), lambda i: (0, i, 0)),
        pl.BlockSpec((1, NC, H), lambda i: (0, 0, 0)),
        pl.BlockSpec((RB, NC), lambda i: (i, 0)),
        pl.BlockSpec((1, 1, RB), lambda i: (i, 0, 0)),
        pl.BlockSpec((D, 12), lambda i: (0, 0)),
        pl.BlockSpec((1, 12), lambda i: (0, 0)),
    ],
    out_specs=pl.BlockSpec((G, 12), lambda i: (0, 0)),
    out_shape=jax.ShapeDtypeStruct((G, 12), jnp.float32),
    scratch_shapes=[
        pltpu.VMEM((G, D), jnp.float32),
        pltpu.VMEM((1, G), jnp.float32),
    ],
)


def kernel(x, W1, b1, W2, b2, Wl, bl, edge_index, batch):
    src = edge_index[0].astype(jnp.int32)
    dst = edge_index[1].astype(jnp.int32)
    src2 = jnp.concatenate([src, src + N])
    batch3 = batch.astype(jnp.int32).reshape(N_RB, 1, RB)
    b1r = b1.reshape(1, NC, H)
    b2r = b2.reshape(1, NC, H)
    W2r = W2.reshape(1, NC, H, D)
    blr = bl.reshape(1, 12)

    deg_flat = _deg_call(dst)
    deg = jnp.stack([deg_flat[:N], deg_flat[NP:NP + N]], axis=1)
    t1 = _mm1_call(x, W1, deg)
    acc1 = _spmm_call(t1.reshape(NC * N, H), src2, dst)
    t2 = _mm2_call(acc1, W2r, b1r, deg)
    acc2 = _spmm_call(t2.reshape(NC * N, H), src2, dst)
    return _pool_call(acc2, b2r, deg, batch3, Wl, blr)


# speedup vs baseline: 17.4385x; 2.0599x over previous
import functools

import jax
import jax.numpy as jnp
from jax import lax
from jax.experimental import pallas as pl
from jax.experimental.pallas import tpu as pltpu
from jax.experimental.pallas import tpu_sc as plsc

N = 10000
E = 160000
D = 256
H = 128
G = 64
NC = 2
NS = 16
R_MAIN = 624
R_EXTRA_OFF = NS * R_MAIN
R_EXTRA = N - R_EXTRA_OFF
EDGES_PER_TILE = E // NS
CH = 80
N_CHUNKS = EDGES_PER_TILE // CH
N_PHASES = 5
PH_CHUNKS = N_CHUNKS // N_PHASES
DEG_EDGES_PER_SC = E // NC
DEG_EDGES_PER_TILE = DEG_EDGES_PER_SC // NS
DCH = 40
N_DEG_CHUNKS = DEG_EDGES_PER_TILE // DCH
RB = 2000
N_RB = N // RB

_sc_mesh = plsc.VectorSubcoreMesh(core_axis_name="c", subcore_axis_name="s")


NP = 10240
NPH = NP // 2
DEG_E_TILE = E // (NC * NS)
COLS = 640


def _deg_body(dst_hbm, deg_out, dstbuf_v, hist_v, tmp_v, red_v, hist_sh):
    c = lax.axis_index("c")
    s = lax.axis_index("s")
    wid = c * NS + s

    def initz(i, _):
        hist_v[pl.ds(i * 16, 16)] = jnp.zeros((16,), jnp.float32)
        return _

    lax.fori_loop(0, NP // 16, initz, None)

    dstbuf_v[pl.ds((DEG_E_TILE // 16) * 16, 16)] = jnp.full((16,), N, jnp.int32)
    pltpu.sync_copy(dst_hbm.at[pl.ds(wid * DEG_E_TILE, DEG_E_TILE)],
                    dstbuf_v.at[pl.ds(0, DEG_E_TILE)])
    ones16 = jnp.ones((16,), jnp.float32)

    def scat(j, _):
        idx = dstbuf_v[pl.ds(j * 16, 16)]
        plsc.addupdate_scatter(hist_v, [idx], ones16)
        return _

    lax.fori_loop(0, DEG_E_TILE // 16 + 1, scat, None)

    for p in range(2):
        pltpu.sync_copy(hist_v.at[pl.ds(p * NPH, NPH)], hist_sh.at[s, 0])
        plsc.subcore_barrier()

        @pl.when(s // 8 == p)
        def _():
            def initr(i, _):
                red_v[pl.ds(i * 16, 16)] = jnp.zeros((16,), jnp.float32)
                return _

            lax.fori_loop(0, COLS // 16, initr, None)

            def redk(k, _):
                pltpu.sync_copy(
                    hist_sh.at[k, 0, pl.ds((s % 8) * COLS, COLS)], tmp_v)

                def add16(i, __):
                    red_v[pl.ds(i * 16, 16)] += tmp_v[pl.ds(i * 16, 16)]
                    return __

                lax.fori_loop(0, COLS // 16, add16, None)
                return _

            lax.fori_loop(0, NS, redk, None)
            pltpu.sync_copy(red_v,
                            deg_out.at[pl.ds(c * NP + s * COLS, COLS)])

        plsc.subcore_barrier()


_deg_call = pl.kernel(
    _deg_body,
    out_type=jax.ShapeDtypeStruct((NC * NP,), jnp.float32),
    mesh=_sc_mesh,
    scratch_types=[
        pltpu.VMEM((DEG_E_TILE + 8,), jnp.int32),
        pltpu.VMEM((NP,), jnp.float32),
        pltpu.VMEM((COLS,), jnp.float32),
        pltpu.VMEM((COLS,), jnp.float32),
        pltpu.VMEM_SHARED((NS, 1, NPH), jnp.float32),
    ],
    compiler_params=pltpu.CompilerParams(needs_layout_passes=False),
)


def _spmm_body(tbl_hbm, src3_hbm, dst3_hbm, out_hbm, src_v, dst_v,
               rows0_v, rows1_v, sem0, sem1, acc_sh):
    c = lax.axis_index("c")
    s = lax.axis_index("s")

    row0 = s * R_MAIN
    pltpu.sync_copy(tbl_hbm.at[pl.ds(c * N + row0, R_MAIN)],
                    acc_sh.at[pl.ds(row0, R_MAIN)])

    @pl.when(s == NS - 1)
    def _():
        pltpu.sync_copy(tbl_hbm.at[pl.ds(c * N + R_EXTRA_OFF, R_EXTRA)],
                        acc_sh.at[pl.ds(R_EXTRA_OFF, R_EXTRA)])

    plsc.subcore_barrier()

    for ph in range(N_PHASES):
        cbase = ph * PH_CHUNKS
        pltpu.sync_copy(
            src3_hbm.at[pl.ds(c * E + s * EDGES_PER_TILE + cbase * CH,
                              PH_CHUNKS * CH)], src_v)
        pltpu.sync_copy(
            dst3_hbm.at[pl.ds(s * N_CHUNKS + cbase, PH_CHUNKS)], dst_v)

        def g(j, buf, sem):
            return pltpu.async_copy(
                tbl_hbm.at[src_v.at[pl.ds(j * CH, CH)]], buf, sem)

        def gwait(j, buf, sem):
            pltpu.make_async_copy(
                tbl_hbm.at[src_v.at[pl.ds(j * CH, CH)]], buf,
                sem).wait()

        def scat(j, buf):
            pltpu.sync_copy(buf, acc_sh.at[dst_v.at[j, 0]], add=True)

        g(0, rows0_v, sem0)
        g(1, rows1_v, sem1)

        def pair(jj, _):
            j0 = 2 * jj
            j1 = j0 + 1
            gwait(j0, rows0_v, sem0)
            scat(j0, rows0_v)

            @pl.when(j0 + 2 < PH_CHUNKS)
            def _():
                g(j0 + 2, rows0_v, sem0)

            gwait(j1, rows1_v, sem1)
            scat(j1, rows1_v)

            @pl.when(j1 + 2 < PH_CHUNKS)
            def _():
                g(j1 + 2, rows1_v, sem1)

            return _

        lax.fori_loop(0, PH_CHUNKS // 2, pair, None)
        gwait(PH_CHUNKS - 1, rows0_v, sem0)
        scat(PH_CHUNKS - 1, rows0_v)

    plsc.subcore_barrier()
    pltpu.sync_copy(acc_sh.at[pl.ds(row0, R_MAIN)],
                    out_hbm.at[c, pl.ds(row0, R_MAIN)])

    @pl.when(s == NS - 1)
    def _():
        pltpu.sync_copy(acc_sh.at[pl.ds(R_EXTRA_OFF, R_EXTRA)],
                        out_hbm.at[c, pl.ds(R_EXTRA_OFF, R_EXTRA)])


_spmm_call = pl.kernel(
    _spmm_body,
    out_type=jax.ShapeDtypeStruct((NC, N, H), jnp.float32),
    mesh=_sc_mesh,
    scratch_types=[
        pltpu.VMEM((PH_CHUNKS * CH,), jnp.int32),
        pltpu.VMEM((PH_CHUNKS, 1, CH), jnp.int32),
        pltpu.VMEM((CH, H), jnp.float32),
        pltpu.VMEM((CH, H), jnp.float32),
        pltpu.SemaphoreType.DMA,
        pltpu.SemaphoreType.DMA,
        pltpu.VMEM_SHARED((N, H), jnp.float32),
    ],
)


def _dinv_block(deg_ref):
    degsum = deg_ref[:, 0:1] + deg_ref[:, 1:2] + 1.0
    return lax.rsqrt(degsum)


def _mm1_body(x_ref, w_ref, deg_ref, o_ref):
    dinv = _dinv_block(deg_ref)
    t = jnp.dot(x_ref[...], w_ref[...], preferred_element_type=jnp.float32)
    o_ref[0] = t * dinv


_mm1_call = pl.pallas_call(
    _mm1_body,
    grid=(N_RB, 2),
    in_specs=[
        pl.BlockSpec((RB, D), lambda i, j: (i, 0)),
        pl.BlockSpec((D, H), lambda i, j: (0, j)),
        pl.BlockSpec((RB, NC), lambda i, j: (i, 0)),
    ],
    out_specs=pl.BlockSpec((1, RB, H), lambda i, j: (j, i, 0)),
    out_shape=jax.ShapeDtypeStruct((NC, N, H), jnp.float32),
)


def _mm2_body(acc_ref, w_ref, b_ref, deg_ref, o_ref):
    dinv = _dinv_block(deg_ref)
    h1a = jax.nn.relu(acc_ref[0] * dinv + b_ref[0, 0][None, :])
    h1b = jax.nn.relu(acc_ref[1] * dinv + b_ref[0, 1][None, :])
    t = (jnp.dot(h1a, w_ref[0, 0], preferred_element_type=jnp.float32)
         + jnp.dot(h1b, w_ref[0, 1], preferred_element_type=jnp.float32))
    o_ref[0] = t * dinv


_mm2_call = pl.pallas_call(
    _mm2_body,
    grid=(N_RB, 2),
    in_specs=[
        pl.BlockSpec((NC, RB, H), lambda i, j: (0, i, 0)),
        pl.BlockSpec((1, NC, H, H), lambda i, j: (0, 0, 0, j)),
        pl.BlockSpec((1, NC, H), lambda i, j: (0, 0, 0)),
        pl.BlockSpec((RB, NC), lambda i, j: (i, 0)),
    ],
    out_specs=pl.BlockSpec((1, RB, H), lambda i, j: (j, i, 0)),
    out_shape=jax.ShapeDtypeStruct((NC, N, H), jnp.float32),
)


def _pool_body(acc_ref, b_ref, deg_ref, batch_ref, wl_ref, bl_ref, o_ref,
               pooled_acc, cnt_acc):
    i = pl.program_id(0)

    @pl.when(i == 0)
    def _():
        pooled_acc[...] = jnp.zeros_like(pooled_acc)
        cnt_acc[...] = jnp.zeros_like(cnt_acc)

    dinv = _dinv_block(deg_ref)
    h2a = jax.nn.relu(acc_ref[0] * dinv + b_ref[0, 0][None, :])
    h2b = jax.nn.relu(acc_ref[1] * dinv + b_ref[0, 1][None, :])
    h2 = jnp.concatenate([h2a, h2b], axis=1)
    bvec = batch_ref[0, 0, :]
    iota = lax.broadcasted_iota(jnp.int32, (RB, G), 1)
    onehot = (bvec[:, None] == iota).astype(jnp.float32)
    pooled_acc[...] += lax.dot_general(
        onehot, h2, (((0,), (0,)), ((), ())),
        preferred_element_type=jnp.float32)
    cnt_acc[...] += jnp.sum(onehot, axis=0, keepdims=True)

    @pl.when(i == N_RB - 1)
    def _():
        cnt = jnp.maximum(cnt_acc[0, :], 1.0)
        pooled = pooled_acc[...] / cnt[:, None]
        o_ref[...] = (jnp.dot(pooled, wl_ref[...],
                              preferred_element_type=jnp.float32)
                      + bl_ref[0, :][None, :])


_pool_call = pl.pallas_call(
    _pool_body,
    grid=(N_RB,),
    in_specs=[
        pl.BlockSpec((NC, RB, H), lambda i: (0, i, 0)),
        pl.BlockSpec((1, NC, H), lambda i: (0, 0, 0)),
        pl.BlockSpec((RB, NC), lambda i: (i, 0)),
        pl.BlockSpec((1, 1, RB), lambda i: (i, 0, 0)),
        pl.BlockSpec((D, 12), lambda i: (0, 0)),
        pl.BlockSpec((1, 12), lambda i: (0, 0)),
    ],
    out_specs=pl.BlockSpec((G, 12), lambda i: (0, 0)),
    out_shape=jax.ShapeDtypeStruct((G, 12), jnp.float32),
    scratch_shapes=[
        pltpu.VMEM((G, D), jnp.float32),
        pltpu.VMEM((1, G), jnp.float32),
    ],
)


def kernel(x, W1, b1, W2, b2, Wl, bl, edge_index, batch):
    src = edge_index[0].astype(jnp.int32)
    dst = edge_index[1].astype(jnp.int32)
    src3 = jnp.concatenate([src, src + N])
    dst3 = dst.reshape(NS * N_CHUNKS, 1, CH)
    batch3 = batch.astype(jnp.int32).reshape(N_RB, 1, RB)
    b1r = b1.reshape(1, NC, H)
    b2r = b2.reshape(1, NC, H)
    W2r = W2.reshape(1, NC, H, D)
    blr = bl.reshape(1, 12)

    deg_flat = _deg_call(dst)
    deg = jnp.stack([deg_flat[:N], deg_flat[NP:NP + N]], axis=1)
    t1 = _mm1_call(x, W1, deg)
    acc1 = _spmm_call(t1.reshape(NC * N, H), src3, dst3)
    t2 = _mm2_call(acc1, W2r, b1r, deg)
    acc2 = _spmm_call(t2.reshape(NC * N, H), src3, dst3)
    return _pool_call(acc2, b2r, deg, batch3, Wl, blr)
